# baseline (speedup 1.0000x reference)
import jax
import jax.numpy as jnp
from jax import lax
from jax.experimental import pallas as pl
from jax.experimental.pallas import tpu as pltpu

_NUM_TOKENS = 32768
_D_MODEL = 1024
_NUM_EXPERTS = 64
_ROWS = 256
_COLS = 128
_NCHUNK = 8
_CHUNK_ROWS = _NUM_TOKENS // _NCHUNK


def _fused_body(x_hbm, tag_ref, load_ref, xout_hbm, cap_ref, sems):
    for c in range(_NCHUNK):
        pltpu.make_async_copy(
            x_hbm.at[pl.ds(c * _CHUNK_ROWS, _CHUNK_ROWS), :],
            xout_hbm.at[pl.ds(c * _CHUNK_ROWS, _CHUNK_ROWS), :],
            sems.at[c],
        ).start()

    tags = tag_ref[...]
    load = load_ref[0, 0]
    for e in range(_NUM_EXPERTS):
        cnt = jnp.sum(jnp.where(tags == e, 1, 0))
        cap_ref[e] = jnp.minimum(cnt, load)

    for c in range(_NCHUNK):
        pltpu.make_async_copy(
            x_hbm.at[pl.ds(c * _CHUNK_ROWS, _CHUNK_ROWS), :],
            xout_hbm.at[pl.ds(c * _CHUNK_ROWS, _CHUNK_ROWS), :],
            sems.at[c],
        ).wait()


@jax.jit
def _fused(x, tag2d, load_arr):
    return pl.pallas_call(
        _fused_body,
        in_specs=[
            pl.BlockSpec(memory_space=pl.ANY),
            pl.BlockSpec(memory_space=pltpu.VMEM),
            pl.BlockSpec(memory_space=pltpu.SMEM),
        ],
        out_specs=[
            pl.BlockSpec(memory_space=pl.ANY),
            pl.BlockSpec(memory_space=pltpu.SMEM),
        ],
        out_shape=[
            jax.ShapeDtypeStruct((_NUM_TOKENS, _D_MODEL), jnp.float32),
            jax.ShapeDtypeStruct((_NUM_EXPERTS,), jnp.int32),
        ],
        scratch_shapes=[pltpu.SemaphoreType.DMA((_NCHUNK,))],
    )(x, tag2d, load_arr)


def kernel(x, tag, load):
    tag2d = tag.reshape(_ROWS, _COLS)
    load_arr = jnp.full((1, 1), load, dtype=jnp.int32)
    x_out, capacity = _fused(x, tag2d, load_arr)
    return (x_out, tag, capacity)


# speedup vs baseline: 47.7761x; 47.7761x over previous
import jax
import jax.numpy as jnp
from jax.experimental import pallas as pl
from jax.experimental.pallas import tpu as pltpu

_NUM_TOKENS = 32768
_D_MODEL = 1024
_NUM_EXPERTS = 64
_ROWS = 256
_COLS = 128
_GRID = 16
_BLOCK_ROWS = _NUM_TOKENS // _GRID


def _fused_body(x_ref, tag_ref, load_ref, xout_ref, cap_ref):
    @pl.when(pl.program_id(0) == 0)
    def _():
        tags = tag_ref[...]
        load = load_ref[0, 0]
        for e in range(_NUM_EXPERTS):
            cnt = jnp.sum(jnp.where(tags == e, 1, 0))
            cap_ref[e] = jnp.minimum(cnt, load)

    xout_ref[...] = x_ref[...]


@jax.jit
def _fused(x, tag2d, load_arr):
    return pl.pallas_call(
        _fused_body,
        grid=(_GRID,),
        in_specs=[
            pl.BlockSpec((_BLOCK_ROWS, _D_MODEL), lambda i: (i, 0)),
            pl.BlockSpec((_ROWS, _COLS), lambda i: (0, 0)),
            pl.BlockSpec(memory_space=pltpu.SMEM),
        ],
        out_specs=[
            pl.BlockSpec((_BLOCK_ROWS, _D_MODEL), lambda i: (i, 0)),
            pl.BlockSpec(memory_space=pltpu.SMEM),
        ],
        out_shape=[
            jax.ShapeDtypeStruct((_NUM_TOKENS, _D_MODEL), jnp.float32),
            jax.ShapeDtypeStruct((_NUM_EXPERTS,), jnp.int32),
        ],
    )(x, tag2d, load_arr)


def kernel(x, tag, load):
    tag2d = tag.reshape(_ROWS, _COLS)
    load_arr = jnp.full((1, 1), load, dtype=jnp.int32)
    x_out, capacity = _fused(x, tag2d, load_arr)
    return (x_out, tag, capacity)
